# baseline (speedup 1.0000x reference)
import jax
import jax.numpy as jnp
from jax import lax
from jax.experimental import pallas as pl
from jax.experimental.pallas import tpu as pltpu, tpu_sc as plsc

_NUM_NODES = 10000
_DIM = 128
_NUM_EDGES = 320000

_info = plsc.get_sparse_core_info()
_NC, _NS, _L = _info.num_cores, _info.num_subcores, _info.num_lanes
_NW = _NC * _NS
_EPW = _NUM_EDGES // _NW
_C = 80
_NCHUNK = _EPW // _C
_G = _C // _L
_NSLC = _DIM // _L
_NBUF = 5


def _sc_body(h_hbm, src_hbm, dst_hbm, out_hbm,
             sidx, didx, srows, drows, obuf, gsem, osem):
    wid = lax.axis_index("s") * _NC + lax.axis_index("c")
    wbase = wid * _EPW

    pltpu.sync_copy(src_hbm.at[pl.ds(wbase, _EPW)], sidx)
    pltpu.sync_copy(dst_hbm.at[pl.ds(wbase, _EPW)], didx)

    def issue_gather(c, k):
        pltpu.async_copy(h_hbm.at[sidx.at[pl.ds(c * _C, _C)]],
                         srows.at[k], gsem.at[k, 0])
        pltpu.async_copy(h_hbm.at[didx.at[pl.ds(c * _C, _C)]],
                         drows.at[k], gsem.at[k, 1])

    def wait_gather(k):
        pltpu.make_async_copy(h_hbm.at[sidx.at[pl.ds(0, _C)]],
                              srows.at[k], gsem.at[k, 0]).wait()
        pltpu.make_async_copy(h_hbm.at[didx.at[pl.ds(0, _C)]],
                              drows.at[k], gsem.at[k, 1]).wait()

    for k in range(_NBUF):
        issue_gather(k, k)

    lanes = lax.iota(jnp.int32, _L)

    def iter_body(i, carry):
        for k in range(_NBUF):
            c = i * _NBUF + k
            wait_gather(k)
            sr = srows.at[k]
            dr = drows.at[k]

            @pl.when(i > 0)
            def _():
                pltpu.make_async_copy(
                    obuf.at[pl.ds(k * (_C + _L), _C)], out_hbm.at[pl.ds(wbase, _C)],
                    osem.at[k]
                ).wait()

            def half_body(hh, dots):
                odd = (hh & 1) == 1
                base_lane = (hh & 1) * 8
                for j in range(8):
                    e = hh * 8 + j
                    acc = sr[e, pl.ds(0, _L)] * dr[e, pl.ds(0, _L)]
                    for s in range(1, _NSLC):
                        acc = acc + sr[e, pl.ds(s * _L, _L)] * dr[e, pl.ds(s * _L, _L)]
                    half = acc + jnp.flip(acc, axis=0)
                    h = [half[m] for m in range(8)]
                    d = ((h[0] + h[1]) + (h[2] + h[3])) + \
                        ((h[4] + h[5]) + (h[6] + h[7]))
                    dots = jnp.where(lanes == base_lane + j, d, dots)

                @pl.when(odd)
                def _():
                    obuf[pl.ds(k * (_C + _L) + (hh >> 1) * _L, _L)] = (
                        1.0 / (1.0 + jnp.exp(-dots)))

                return jnp.where(odd, jnp.zeros((_L,), jnp.float32), dots)

            lax.fori_loop(0, 2 * _G, half_body, jnp.zeros((_L,), jnp.float32))

            pltpu.async_copy(obuf.at[pl.ds(k * (_C + _L), _C)],
                             out_hbm.at[pl.ds(wbase + c * _C, _C)], osem.at[k])

            @pl.when(c + _NBUF < _NCHUNK)
            def _():
                issue_gather(c + _NBUF, k)
        return carry

    lax.fori_loop(0, _NCHUNK // _NBUF, iter_body, 0)

    for k in range(_NBUF):
        pltpu.make_async_copy(obuf.at[pl.ds(k * (_C + _L), _C)],
                              out_hbm.at[pl.ds(wbase, _C)],
                              osem.at[k]).wait()


def kernel(h, edge_index):
    src = edge_index[0].astype(jnp.int32)
    dst = edge_index[1].astype(jnp.int32)
    mesh = plsc.VectorSubcoreMesh(core_axis_name="c", subcore_axis_name="s")
    k = pl.kernel(
        _sc_body,
        out_type=jax.ShapeDtypeStruct((_NUM_EDGES,), jnp.float32),
        mesh=mesh,
        scratch_types=[
            pltpu.VMEM((_EPW,), jnp.int32),
            pltpu.VMEM((_EPW,), jnp.int32),
            pltpu.VMEM((_NBUF, _C, _DIM), jnp.float32),
            pltpu.VMEM((_NBUF, _C, _DIM), jnp.float32),
            pltpu.VMEM((_NBUF * (_C + _L),), jnp.float32),
            pltpu.SemaphoreType.DMA((_NBUF, 2)),
            pltpu.SemaphoreType.DMA((_NBUF,)),
        ],
    )
    return k(h, src, dst)


# speedup vs baseline: 1.0740x; 1.0740x over previous
import jax
import jax.numpy as jnp
from jax import lax
from jax.experimental import pallas as pl
from jax.experimental.pallas import tpu as pltpu, tpu_sc as plsc

_NUM_NODES = 10000
_DIM = 128
_NUM_EDGES = 320000

_info = plsc.get_sparse_core_info()
_NC, _NS, _L = _info.num_cores, _info.num_subcores, _info.num_lanes
_NW = _NC * _NS
_EPW = _NUM_EDGES // _NW
_C = 80
_NCHUNK = _EPW // _C
_G = _C // _L
_NSLC = _DIM // _L
_NBUF = 5


def _sc_body(h_hbm, src_hbm, dst_hbm, out_hbm,
             sidx, didx, srows, drows, obuf, gsem, osem):
    wid = lax.axis_index("s") * _NC + lax.axis_index("c")
    wbase = wid * _EPW

    pltpu.sync_copy(src_hbm.at[pl.ds(wbase, _EPW)], sidx)
    pltpu.sync_copy(dst_hbm.at[pl.ds(wbase, _EPW)], didx)

    def issue_gather(c, k):
        pltpu.async_copy(h_hbm.at[sidx.at[pl.ds(c * _C, _C)]],
                         srows.at[k], gsem.at[k, 0])
        pltpu.async_copy(h_hbm.at[didx.at[pl.ds(c * _C, _C)]],
                         drows.at[k], gsem.at[k, 1])

    def wait_gather(k):
        pltpu.make_async_copy(h_hbm.at[sidx.at[pl.ds(0, _C)]],
                              srows.at[k], gsem.at[k, 0]).wait()
        pltpu.make_async_copy(h_hbm.at[didx.at[pl.ds(0, _C)]],
                              drows.at[k], gsem.at[k, 1]).wait()

    for k in range(_NBUF):
        issue_gather(k, k)

    lanes = lax.iota(jnp.int32, _L)

    def iter_body(i, carry):
        for k in range(_NBUF):
            c = i * _NBUF + k
            wait_gather(k)
            sr = srows.at[k]
            dr = drows.at[k]

            @pl.when(i > 0)
            def _():
                pltpu.make_async_copy(
                    obuf.at[pl.ds(k * (_C + _L), _C)], out_hbm.at[pl.ds(wbase, _C)],
                    osem.at[k]
                ).wait()

            def half_body(hh, dots):
                odd = (hh & 1) == 1
                base_lane = (hh & 1) * 8
                for j in range(8):
                    e = hh * 8 + j
                    acc = sr[e, pl.ds(0, _L)] * dr[e, pl.ds(0, _L)]
                    for s in range(1, _NSLC):
                        acc = acc + sr[e, pl.ds(s * _L, _L)] * dr[e, pl.ds(s * _L, _L)]
                    half = acc + jnp.flip(acc, axis=0)
                    h = [half[m] for m in range(8)]
                    d = ((h[0] + h[1]) + (h[2] + h[3])) + \
                        ((h[4] + h[5]) + (h[6] + h[7]))
                    dots = jnp.where(lanes == base_lane + j, d, dots)

                @pl.when(odd)
                def _():
                    obuf[pl.ds(k * (_C + _L) + (hh >> 1) * _L, _L)] = dots

                return jnp.where(odd, jnp.zeros((_L,), jnp.float32), dots)

            lax.fori_loop(0, 2 * _G, half_body, jnp.zeros((_L,), jnp.float32))

            for g in range(_G):
                v = obuf[pl.ds(k * (_C + _L) + g * _L, _L)]
                obuf[pl.ds(k * (_C + _L) + g * _L, _L)] = (
                    1.0 / (1.0 + jnp.exp(-v)))

            pltpu.async_copy(obuf.at[pl.ds(k * (_C + _L), _C)],
                             out_hbm.at[pl.ds(wbase + c * _C, _C)], osem.at[k])

            @pl.when(c + _NBUF < _NCHUNK)
            def _():
                issue_gather(c + _NBUF, k)
        return carry

    lax.fori_loop(0, _NCHUNK // _NBUF, iter_body, 0)

    for k in range(_NBUF):
        pltpu.make_async_copy(obuf.at[pl.ds(k * (_C + _L), _C)],
                              out_hbm.at[pl.ds(wbase, _C)],
                              osem.at[k]).wait()


def kernel(h, edge_index):
    src = edge_index[0].astype(jnp.int32)
    dst = edge_index[1].astype(jnp.int32)
    mesh = plsc.VectorSubcoreMesh(core_axis_name="c", subcore_axis_name="s")
    k = pl.kernel(
        _sc_body,
        out_type=jax.ShapeDtypeStruct((_NUM_EDGES,), jnp.float32),
        mesh=mesh,
        scratch_types=[
            pltpu.VMEM((_EPW,), jnp.int32),
            pltpu.VMEM((_EPW,), jnp.int32),
            pltpu.VMEM((_NBUF, _C, _DIM), jnp.float32),
            pltpu.VMEM((_NBUF, _C, _DIM), jnp.float32),
            pltpu.VMEM((_NBUF * (_C + _L),), jnp.float32),
            pltpu.SemaphoreType.DMA((_NBUF, 2)),
            pltpu.SemaphoreType.DMA((_NBUF,)),
        ],
    )
    return k(h, src, dst)


# speedup vs baseline: 1.0792x; 1.0048x over previous
import jax
import jax.numpy as jnp
from jax import lax
from jax.experimental import pallas as pl
from jax.experimental.pallas import tpu as pltpu, tpu_sc as plsc

_NUM_NODES = 10000
_DIM = 128
_NUM_EDGES = 320000

_info = plsc.get_sparse_core_info()
_NC, _NS, _L = _info.num_cores, _info.num_subcores, _info.num_lanes
_NW = _NC * _NS
_EPW = _NUM_EDGES // _NW
_C = 80
_NCHUNK = _EPW // _C
_G = _C // _L
_NSLC = _DIM // _L
_NBUF = 5


def _sc_body(h_hbm, src_hbm, dst_hbm, out_hbm,
             sidx, didx, srows, drows, obuf, gsem, osem):
    wid = lax.axis_index("s") * _NC + lax.axis_index("c")
    wbase = wid * _EPW

    pltpu.sync_copy(src_hbm.at[pl.ds(wbase, _EPW)], sidx)
    pltpu.sync_copy(dst_hbm.at[pl.ds(wbase, _EPW)], didx)

    def issue_gather(c, k):
        pltpu.async_copy(h_hbm.at[sidx.at[pl.ds(c * _C, _C)]],
                         srows.at[k], gsem.at[k, 0])
        pltpu.async_copy(h_hbm.at[didx.at[pl.ds(c * _C, _C)]],
                         drows.at[k], gsem.at[k, 1])

    def wait_gather(k):
        pltpu.make_async_copy(h_hbm.at[sidx.at[pl.ds(0, _C)]],
                              srows.at[k], gsem.at[k, 0]).wait()
        pltpu.make_async_copy(h_hbm.at[didx.at[pl.ds(0, _C)]],
                              drows.at[k], gsem.at[k, 1]).wait()

    for k in range(_NBUF):
        issue_gather(k, k)

    lanes = lax.iota(jnp.int32, _L)

    def iter_body(i, carry):
        for k in range(_NBUF):
            c = i * _NBUF + k
            wait_gather(k)
            sr = srows.at[k]
            dr = drows.at[k]

            @pl.when(i > 0)
            def _():
                pltpu.make_async_copy(
                    obuf.at[pl.ds(k * (_C + _L), _C)], out_hbm.at[pl.ds(wbase, _C)],
                    osem.at[k]
                ).wait()

            def half_body(hh, hcarry):
                dots = jnp.zeros((_L,), jnp.float32)
                for j in range(8):
                    e = hh * 8 + j
                    acc = sr[e, pl.ds(0, _L)] * dr[e, pl.ds(0, _L)]
                    for s in range(1, _NSLC):
                        acc = acc + sr[e, pl.ds(s * _L, _L)] * dr[e, pl.ds(s * _L, _L)]
                    half = acc + jnp.flip(acc, axis=0)
                    h = [half[m] for m in range(8)]
                    d = ((h[0] + h[1]) + (h[2] + h[3])) + \
                        ((h[4] + h[5]) + (h[6] + h[7]))
                    dots = jnp.where(lanes == j, d, dots)

                obuf[pl.ds(k * (_C + _L) + hh * 8, _L)] = dots
                return hcarry

            lax.fori_loop(0, 2 * _G, half_body, 0)

            for g in range(_G):
                v = obuf[pl.ds(k * (_C + _L) + g * _L, _L)]
                obuf[pl.ds(k * (_C + _L) + g * _L, _L)] = (
                    1.0 / (1.0 + jnp.exp(-v)))

            pltpu.async_copy(obuf.at[pl.ds(k * (_C + _L), _C)],
                             out_hbm.at[pl.ds(wbase + c * _C, _C)], osem.at[k])

            @pl.when(c + _NBUF < _NCHUNK)
            def _():
                issue_gather(c + _NBUF, k)
        return carry

    lax.fori_loop(0, _NCHUNK // _NBUF, iter_body, 0)

    for k in range(_NBUF):
        pltpu.make_async_copy(obuf.at[pl.ds(k * (_C + _L), _C)],
                              out_hbm.at[pl.ds(wbase, _C)],
                              osem.at[k]).wait()


def kernel(h, edge_index):
    src = edge_index[0].astype(jnp.int32)
    dst = edge_index[1].astype(jnp.int32)
    mesh = plsc.VectorSubcoreMesh(core_axis_name="c", subcore_axis_name="s")
    k = pl.kernel(
        _sc_body,
        out_type=jax.ShapeDtypeStruct((_NUM_EDGES,), jnp.float32),
        mesh=mesh,
        scratch_types=[
            pltpu.VMEM((_EPW,), jnp.int32),
            pltpu.VMEM((_EPW,), jnp.int32),
            pltpu.VMEM((_NBUF, _C, _DIM), jnp.float32),
            pltpu.VMEM((_NBUF, _C, _DIM), jnp.float32),
            pltpu.VMEM((_NBUF * (_C + _L),), jnp.float32),
            pltpu.SemaphoreType.DMA((_NBUF, 2)),
            pltpu.SemaphoreType.DMA((_NBUF,)),
        ],
    )
    return k(h, src, dst)
